# baseline (speedup 1.0000x reference)
import functools

import jax
import jax.numpy as jnp
import numpy as np
from jax import lax
from jax.experimental import pallas as pl
from jax.experimental.pallas import tpu as pltpu
from jax.experimental.pallas import tpu_sc as plsc

N = 50000
E = 800000
D = 64
HID = 64
HEADS = 4
DH = 16
SCALE = DH ** -0.5

NCORE = 2
NSUB = 16

GB = 128
SB = 40
PW = 40
NPT = N // NSUB
HP = jax.lax.Precision.HIGHEST

_SEL = np.kron(np.eye(HEADS, dtype=np.float32), np.ones((DH, 1), np.float32))


def _dense_in_body(x_ref, w_ref, b_ref, o_ref):
    o_ref[...] = jnp.maximum(x_ref[...] @ w_ref[...] + b_ref[...], 0.0)


def _dense_in(x, w, b):
    BN = 2000
    return pl.pallas_call(
        _dense_in_body,
        grid=(N // BN,),
        in_specs=[
            pl.BlockSpec((BN, D), lambda i: (i, 0)),
            pl.BlockSpec((D, HID), lambda i: (0, 0)),
            pl.BlockSpec((1, HID), lambda i: (0, 0)),
        ],
        out_specs=pl.BlockSpec((BN, HID), lambda i: (i, 0)),
        out_shape=jax.ShapeDtypeStruct((N, HID), jnp.float32),
    )(x, w, b.reshape(1, HID))


def _qkv_body(h_ref, wq_ref, wkv_ref, q_ref, kv_ref):
    h = h_ref[...]
    q_ref[...] = (h @ wq_ref[...]).astype(jnp.bfloat16)
    kv_ref[...] = (h @ wkv_ref[...]).astype(jnp.bfloat16)


def _qkv(h, wq, wkv):
    BN = 2000
    return pl.pallas_call(
        _qkv_body,
        grid=(N // BN,),
        in_specs=[
            pl.BlockSpec((BN, HID), lambda i: (i, 0)),
            pl.BlockSpec((HID, 64), lambda i: (0, 0)),
            pl.BlockSpec((HID, 128), lambda i: (0, 0)),
        ],
        out_specs=[
            pl.BlockSpec((BN, 64), lambda i: (i, 0)),
            pl.BlockSpec((BN, 128), lambda i: (i, 0)),
        ],
        out_shape=[
            jax.ShapeDtypeStruct((N, 64), jnp.bfloat16),
            jax.ShapeDtypeStruct((N, 128), jnp.bfloat16),
        ],
    )(h, wq, wkv)


def _edge_body(qd_ref, kvs_ref, ea_ref, we_ref, o_ref):
    qd = qd_ref[...].astype(jnp.float32)
    kv = kvs_ref[...].astype(jnp.float32)
    ks = kv[:, :64]
    vs = kv[:, 64:]
    m = qd * ks
    be = m.shape[0]
    s = jnp.concatenate(
        [jnp.sum(m[:, h * DH:(h + 1) * DH], axis=1, keepdims=True)
         for h in range(HEADS)], axis=1)
    ea = ea_ref[...].astype(jnp.bfloat16).astype(jnp.float32)
    we = we_ref[...].astype(jnp.bfloat16).astype(jnp.float32)
    eb = (ea[:, 0:1] * we[0:1, :] + ea[:, 1:2] * we[1:2, :]
          + ea[:, 2:3] * we[2:3, :])
    s = s * SCALE + eb
    s = jnp.where(s >= 0.0, s, 0.2 * s)
    p = jnp.exp(s)
    pb = jnp.concatenate(
        [jnp.broadcast_to(p[:, h:h + 1], (be, DH)) for h in range(HEADS)],
        axis=1)
    w = vs * pb
    z = jnp.zeros((be, 4), jnp.float32)
    o_ref[0] = jnp.concatenate([w[:, :32], p, z], axis=1)
    o_ref[1] = jnp.concatenate([w[:, 32:], p, z], axis=1)


def _edge_math(qd, kvs, edge_attr, we):
    BE = 2000
    return pl.pallas_call(
        _edge_body,
        grid=(E // BE,),
        in_specs=[
            pl.BlockSpec((BE, 64), lambda i: (i, 0)),
            pl.BlockSpec((BE, 128), lambda i: (i, 0)),
            pl.BlockSpec((BE, 3), lambda i: (i, 0)),
            pl.BlockSpec((3, HEADS), lambda i: (0, 0)),
        ],
        out_specs=pl.BlockSpec((2, BE, PW), lambda i: (0, i, 0)),
        out_shape=jax.ShapeDtypeStruct((2, E, PW), jnp.float32),
    )(qd, kvs, edge_attr, we)


def _post_body(h_ref, wp_ref, wo_ref, bo_ref, wmh_ref, wma_ref, bm_ref,
               g_ref, b_ref, o_ref):
    wp0 = wp_ref[0]
    wp1 = wp_ref[1]
    ps = jnp.concatenate([wp0[:, 32:34], wp1[:, 34:36]], axis=1)
    den = jnp.maximum(ps, 1e-12)
    bn = den.shape[0]
    den_b = jnp.concatenate(
        [jnp.broadcast_to(den[:, h:h + 1], (bn, DH)) for h in range(HEADS)],
        axis=1)
    agg = jnp.concatenate([wp0[:, :32], wp1[:, :32]], axis=1)
    agg = agg / den_b
    agg2 = agg @ wo_ref[...] + bo_ref[...]
    h = h_ref[...]
    hc = jnp.maximum(h @ wmh_ref[...] + agg2 @ wma_ref[...] + bm_ref[...], 0.0)
    hr = h + hc
    mu = jnp.mean(hr, axis=1, keepdims=True)
    var = jnp.mean((hr - mu) ** 2, axis=1, keepdims=True)
    o_ref[...] = g_ref[...] * (hr - mu) * lax.rsqrt(var + 1e-5) + b_ref[...]


def _post(h, wp, p):
    BN = 2000
    row = lambda a: a.reshape(1, HID)
    return pl.pallas_call(
        _post_body,
        grid=(N // BN,),
        in_specs=[
            pl.BlockSpec((BN, HID), lambda i: (i, 0)),
            pl.BlockSpec((2, BN, PW), lambda i: (0, i, 0)),
            pl.BlockSpec((64, HID), lambda i: (0, 0)),
            pl.BlockSpec((1, HID), lambda i: (0, 0)),
            pl.BlockSpec((HID, HID), lambda i: (0, 0)),
            pl.BlockSpec((HID, HID), lambda i: (0, 0)),
            pl.BlockSpec((1, HID), lambda i: (0, 0)),
            pl.BlockSpec((1, HID), lambda i: (0, 0)),
            pl.BlockSpec((1, HID), lambda i: (0, 0)),
        ],
        out_specs=pl.BlockSpec((BN, HID), lambda i: (i, 0)),
        out_shape=jax.ShapeDtypeStruct((N, HID), jnp.float32),
    )(h, wp, p['Wo'], row(p['bo']), p['Wm'][:HID], p['Wm'][HID:],
      row(p['bm']), row(p['g']), row(p['b']))


def _head_body(h_ref, w1_ref, b1_ref, w2_ref, b2_ref, o_ref):
    h1 = jnp.maximum(h_ref[...] @ w1_ref[...] + b1_ref[...], 0.0)
    o_ref[...] = h1 @ w2_ref[...] + b2_ref[...]


def _head(h, w1, b1, w2, b2):
    BN = 2000
    return pl.pallas_call(
        _head_body,
        grid=(N // BN,),
        in_specs=[
            pl.BlockSpec((BN, HID), lambda i: (i, 0)),
            pl.BlockSpec((HID, HID // 2), lambda i: (0, 0)),
            pl.BlockSpec((1, HID // 2), lambda i: (0, 0)),
            pl.BlockSpec((HID // 2, 1), lambda i: (0, 0)),
            pl.BlockSpec((1, 1), lambda i: (0, 0)),
        ],
        out_specs=pl.BlockSpec((BN, 1), lambda i: (i, 0)),
        out_shape=jax.ShapeDtypeStruct((N, 1), jnp.float32),
    )(h, w1, b1.reshape(1, HID // 2), w2, b2.reshape(1, 1))


_MESH = plsc.VectorSubcoreMesh(core_axis_name="c", subcore_axis_name="s")
_SC_PARAMS = pltpu.CompilerParams(use_tc_tiling_on_sc=False)


@jax.jit
def _sc_gather(q, kv, dst1, src1):
    @functools.partial(
        pl.kernel,
        mesh=_MESH,
        compiler_params=_SC_PARAMS,
        out_type=[
            jax.ShapeDtypeStruct((E, 64), jnp.bfloat16),
            jax.ShapeDtypeStruct((E, 128), jnp.bfloat16),
        ],
    )
    def k(q_hbm, kv_hbm, dst_hbm, src_hbm, qd_out, kvs_out):
        def body(dst_v, src_v, qd_v, kvs_v):
            pltpu.sync_copy(q_hbm.at[dst_v.at[0]], qd_v)
            pltpu.sync_copy(kv_hbm.at[src_v.at[0]], kvs_v)

        pltpu.emit_pipeline(
            body,
            grid=(E // GB,),
            in_specs=[
                pl.BlockSpec((1, GB), lambda i: (0, i)),
                pl.BlockSpec((1, GB), lambda i: (0, i)),
            ],
            out_specs=[
                pl.BlockSpec((GB, 64), lambda i: (i, 0)),
                pl.BlockSpec((GB, 128), lambda i: (i, 0)),
            ],
            core_axis_name=("c", "s"),
            dimension_semantics=(pltpu.PARALLEL,),
        )(dst_hbm, src_hbm, qd_out, kvs_out)

    return k(q, kv, dst1, src1)


@jax.jit
def _sc_scatter(wp, dst2, zeros):
    @functools.partial(
        pl.kernel,
        mesh=_MESH,
        compiler_params=_SC_PARAMS,
        out_type=jax.ShapeDtypeStruct((2, N, PW), jnp.float32),
        scratch_types=[
            pltpu.VMEM_SHARED((N, PW), jnp.float32),
        ],
    )
    def k(wp_hbm, dst_hbm, z_hbm, out_hbm, acc):
        c = lax.axis_index("c")
        s = lax.axis_index("s")
        pltpu.sync_copy(z_hbm.at[pl.ds(s * NPT, NPT)],
                        acc.at[pl.ds(s * NPT, NPT)])
        plsc.subcore_barrier()

        def body(ib_v, wb_v):
            pltpu.sync_copy(wb_v, acc.at[ib_v.at[0]], add=True)

        pltpu.emit_pipeline(
            body,
            grid=(E // SB,),
            in_specs=[
                pl.BlockSpec((1, SB), lambda i: (0, i)),
                pl.BlockSpec((SB, PW), lambda i: (i, 0)),
            ],
            core_axis_name="s",
            dimension_semantics=(pltpu.PARALLEL,),
        )(dst_hbm, wp_hbm.at[c])

        plsc.subcore_barrier()
        pltpu.sync_copy(acc.at[pl.ds(s * NPT, NPT)],
                        out_hbm.at[c].at[pl.ds(s * NPT, NPT)])

    return k(wp, dst2, zeros)


def kernel(x, edge_index, edge_attr, params):
    src1 = edge_index[0].reshape(1, E)
    dst1 = edge_index[1].reshape(1, E)
    zeros = jnp.zeros((N, PW), jnp.float32)

    h = _dense_in(x, params['W_in'], params['b_in'])
    for i in range(3):
        p = params['layers'][i]
        wkv = jnp.concatenate([p['Wk'], p['Wv']], axis=1)
        q, kv = _qkv(h, p['Wq'], wkv)
        qd, kvs = _sc_gather(q, kv, dst1, src1)
        wp = _edge_math(qd, kvs, edge_attr, p['We'])
        acc = _sc_scatter(wp, dst1, zeros)
        h = _post(h, acc, p)
    return _head(h, params['Wh1'], params['bh1'], params['Wh2'], params['bh2'])


# speedup vs baseline: 1.2247x; 1.2247x over previous
import functools

import jax
import jax.numpy as jnp
from jax import lax
from jax.experimental import pallas as pl
from jax.experimental.pallas import tpu as pltpu
from jax.experimental.pallas import tpu_sc as plsc

N = 50000
E = 800000
D = 64
HID = 64
HEADS = 4
DH = 16
SCALE = DH ** -0.5

NCORE = 2
NSUB = 16

GB = 128
SB = 40
PW = 40
NPT = N // NSUB


def _dense_in_body(x_ref, w_ref, b_ref, o_ref):
    o_ref[...] = jnp.maximum(x_ref[...] @ w_ref[...] + b_ref[...], 0.0)


def _dense_in(x, w, b):
    BN = 2000
    return pl.pallas_call(
        _dense_in_body,
        grid=(N // BN,),
        in_specs=[
            pl.BlockSpec((BN, D), lambda i: (i, 0)),
            pl.BlockSpec((D, HID), lambda i: (0, 0)),
            pl.BlockSpec((1, HID), lambda i: (0, 0)),
        ],
        out_specs=pl.BlockSpec((BN, HID), lambda i: (i, 0)),
        out_shape=jax.ShapeDtypeStruct((N, HID), jnp.float32),
    )(x, w, b.reshape(1, HID))


def _qkv_body(h_ref, wq_ref, wkv_ref, q_ref, kv_ref):
    h = h_ref[...]
    q_ref[...] = h @ wq_ref[...]
    kv_ref[...] = h @ wkv_ref[...]


def _qkv(h, wq, wkv):
    BN = 2000
    return pl.pallas_call(
        _qkv_body,
        grid=(N // BN,),
        in_specs=[
            pl.BlockSpec((BN, HID), lambda i: (i, 0)),
            pl.BlockSpec((HID, 64), lambda i: (0, 0)),
            pl.BlockSpec((HID, 128), lambda i: (0, 0)),
        ],
        out_specs=[
            pl.BlockSpec((BN, 64), lambda i: (i, 0)),
            pl.BlockSpec((BN, 128), lambda i: (i, 0)),
        ],
        out_shape=[
            jax.ShapeDtypeStruct((N, 64), jnp.float32),
            jax.ShapeDtypeStruct((N, 128), jnp.float32),
        ],
    )(h, wq, wkv)


def _edge_body(qd_ref, kvs_ref, ea_ref, we_ref, o_ref):
    qd = qd_ref[...]
    kv = kvs_ref[...]
    ks = kv[:, :64]
    vs = kv[:, 64:]
    m = qd * ks
    be = m.shape[0]
    s = jnp.concatenate(
        [jnp.sum(m[:, h * DH:(h + 1) * DH], axis=1, keepdims=True)
         for h in range(HEADS)], axis=1)
    ea = ea_ref[...].astype(jnp.bfloat16).astype(jnp.float32)
    we = we_ref[...].astype(jnp.bfloat16).astype(jnp.float32)
    eb = (ea[:, 0:1] * we[0:1, :] + ea[:, 1:2] * we[1:2, :]
          + ea[:, 2:3] * we[2:3, :])
    s = s * SCALE + eb
    s = jnp.where(s >= 0.0, s, 0.2 * s)
    p = jnp.exp(s)
    pb = jnp.concatenate(
        [jnp.broadcast_to(p[:, h:h + 1], (be, DH)) for h in range(HEADS)],
        axis=1)
    w = vs * pb
    z = jnp.zeros((be, 4), jnp.float32)
    o_ref[0] = jnp.concatenate([w[:, :32], p, z], axis=1)
    o_ref[1] = jnp.concatenate([w[:, 32:], p, z], axis=1)


def _edge_math(qd, kvs, edge_attr, we):
    BE = 2000
    return pl.pallas_call(
        _edge_body,
        grid=(E // BE,),
        in_specs=[
            pl.BlockSpec((BE, 64), lambda i: (i, 0)),
            pl.BlockSpec((BE, 128), lambda i: (i, 0)),
            pl.BlockSpec((BE, 3), lambda i: (i, 0)),
            pl.BlockSpec((3, HEADS), lambda i: (0, 0)),
        ],
        out_specs=pl.BlockSpec((2, BE, PW), lambda i: (0, i, 0)),
        out_shape=jax.ShapeDtypeStruct((2, E, PW), jnp.float32),
    )(qd, kvs, edge_attr, we)


def _post_body(h_ref, wp_ref, wo_ref, bo_ref, wmh_ref, wma_ref, bm_ref,
               g_ref, b_ref, o_ref):
    wp0 = wp_ref[0]
    wp1 = wp_ref[1]
    ps = jnp.concatenate([wp0[:, 32:34], wp1[:, 34:36]], axis=1)
    den = jnp.maximum(ps, 1e-12)
    bn = den.shape[0]
    den_b = jnp.concatenate(
        [jnp.broadcast_to(den[:, h:h + 1], (bn, DH)) for h in range(HEADS)],
        axis=1)
    agg = jnp.concatenate([wp0[:, :32], wp1[:, :32]], axis=1)
    agg = agg / den_b
    agg2 = agg @ wo_ref[...] + bo_ref[...]
    h = h_ref[...]
    hc = jnp.maximum(h @ wmh_ref[...] + agg2 @ wma_ref[...] + bm_ref[...], 0.0)
    hr = h + hc
    mu = jnp.mean(hr, axis=1, keepdims=True)
    var = jnp.mean((hr - mu) ** 2, axis=1, keepdims=True)
    o_ref[...] = g_ref[...] * (hr - mu) * lax.rsqrt(var + 1e-5) + b_ref[...]


def _post(h, wp, p):
    BN = 2000
    row = lambda a: a.reshape(1, HID)
    return pl.pallas_call(
        _post_body,
        grid=(N // BN,),
        in_specs=[
            pl.BlockSpec((BN, HID), lambda i: (i, 0)),
            pl.BlockSpec((2, BN, PW), lambda i: (0, i, 0)),
            pl.BlockSpec((64, HID), lambda i: (0, 0)),
            pl.BlockSpec((1, HID), lambda i: (0, 0)),
            pl.BlockSpec((HID, HID), lambda i: (0, 0)),
            pl.BlockSpec((HID, HID), lambda i: (0, 0)),
            pl.BlockSpec((1, HID), lambda i: (0, 0)),
            pl.BlockSpec((1, HID), lambda i: (0, 0)),
            pl.BlockSpec((1, HID), lambda i: (0, 0)),
        ],
        out_specs=pl.BlockSpec((BN, HID), lambda i: (i, 0)),
        out_shape=jax.ShapeDtypeStruct((N, HID), jnp.float32),
    )(h, wp, p['Wo'], row(p['bo']), p['Wm'][:HID], p['Wm'][HID:],
      row(p['bm']), row(p['g']), row(p['b']))


def _head_body(h_ref, w1_ref, b1_ref, w2_ref, b2_ref, o_ref):
    h1 = jnp.maximum(h_ref[...] @ w1_ref[...] + b1_ref[...], 0.0)
    o_ref[...] = h1 @ w2_ref[...] + b2_ref[...]


def _head(h, w1, b1, w2, b2):
    BN = 2000
    return pl.pallas_call(
        _head_body,
        grid=(N // BN,),
        in_specs=[
            pl.BlockSpec((BN, HID), lambda i: (i, 0)),
            pl.BlockSpec((HID, HID // 2), lambda i: (0, 0)),
            pl.BlockSpec((1, HID // 2), lambda i: (0, 0)),
            pl.BlockSpec((HID // 2, 1), lambda i: (0, 0)),
            pl.BlockSpec((1, 1), lambda i: (0, 0)),
        ],
        out_specs=pl.BlockSpec((BN, 1), lambda i: (i, 0)),
        out_shape=jax.ShapeDtypeStruct((N, 1), jnp.float32),
    )(h, w1, b1.reshape(1, HID // 2), w2, b2.reshape(1, 1))


_MESH = plsc.VectorSubcoreMesh(core_axis_name="c", subcore_axis_name="s")
_SC_PARAMS = pltpu.CompilerParams(use_tc_tiling_on_sc=False)


@jax.jit
def _sc_gather(q, kv, dst1, src1):
    @functools.partial(
        pl.kernel,
        mesh=_MESH,
        compiler_params=_SC_PARAMS,
        out_type=[
            jax.ShapeDtypeStruct((E, 64), jnp.float32),
            jax.ShapeDtypeStruct((E, 128), jnp.float32),
        ],
    )
    def k(q_hbm, kv_hbm, dst_hbm, src_hbm, qd_out, kvs_out):
        def body(dst_v, src_v, qd_v, kvs_v):
            pltpu.sync_copy(q_hbm.at[dst_v.at[0]], qd_v)
            pltpu.sync_copy(kv_hbm.at[src_v.at[0]], kvs_v)

        pltpu.emit_pipeline(
            body,
            grid=(E // GB,),
            in_specs=[
                pl.BlockSpec((1, GB), lambda i: (0, i)),
                pl.BlockSpec((1, GB), lambda i: (0, i)),
            ],
            out_specs=[
                pl.BlockSpec((GB, 64), lambda i: (i, 0)),
                pl.BlockSpec((GB, 128), lambda i: (i, 0)),
            ],
            core_axis_name=("c", "s"),
            dimension_semantics=(pltpu.PARALLEL,),
        )(dst_hbm, src_hbm, qd_out, kvs_out)

    return k(q, kv, dst1, src1)


@jax.jit
def _sc_scatter(wp, dst2, zeros):
    @functools.partial(
        pl.kernel,
        mesh=_MESH,
        compiler_params=_SC_PARAMS,
        out_type=jax.ShapeDtypeStruct((2, N, PW), jnp.float32),
        scratch_types=[
            pltpu.VMEM_SHARED((N, PW), jnp.float32),
        ],
    )
    def k(wp_hbm, dst_hbm, z_hbm, out_hbm, acc):
        c = lax.axis_index("c")
        s = lax.axis_index("s")
        pltpu.sync_copy(z_hbm.at[pl.ds(s * NPT, NPT)],
                        acc.at[pl.ds(s * NPT, NPT)])
        plsc.subcore_barrier()

        def body(ib_v, wb_v):
            pltpu.sync_copy(wb_v, acc.at[ib_v.at[0]], add=True)

        pltpu.emit_pipeline(
            body,
            grid=(E // SB,),
            in_specs=[
                pl.BlockSpec((1, SB), lambda i: (0, i)),
                pl.BlockSpec((SB, PW), lambda i: (i, 0)),
            ],
            core_axis_name="s",
            dimension_semantics=(pltpu.PARALLEL,),
        )(dst_hbm, wp_hbm.at[c])

        plsc.subcore_barrier()
        pltpu.sync_copy(acc.at[pl.ds(s * NPT, NPT)],
                        out_hbm.at[c].at[pl.ds(s * NPT, NPT)])

    return k(wp, dst2, zeros)


def kernel(x, edge_index, edge_attr, params):
    src1 = edge_index[0].reshape(1, E)
    dst1 = edge_index[1].reshape(1, E)
    zeros = jnp.zeros((N, PW), jnp.float32)

    h = _dense_in(x, params['W_in'], params['b_in'])
    for i in range(3):
        p = params['layers'][i]
        wkv = jnp.concatenate([p['Wk'], p['Wv']], axis=1)
        q, kv = _qkv(h, p['Wq'], wkv)
        qd, kvs = _sc_gather(q, kv, dst1, src1)
        wp = _edge_math(qd, kvs, edge_attr, p['We'])
        acc = _sc_scatter(wp, dst1, zeros)
        h = _post(h, acc, p)
    return _head(h, params['Wh1'], params['bh1'], params['Wh2'], params['bh2'])
